# Initial kernel scaffold; baseline (speedup 1.0000x reference)
import jax
import jax.numpy as jnp
from jax.experimental import pallas as pl


def kernel(z, edge_index, edge_attr, emb, We1, be1, We2, be2, Wm1, bm1, Wm2, bm2, Wp1, bp1, Wp2, bp2):
    raise NotImplementedError("write your pallas kernel here")


# speedup vs baseline: 2.3976x; 2.3976x over previous
import functools

import jax
import jax.numpy as jnp
import numpy as np
from jax import lax
from jax.experimental import pallas as pl
from jax.experimental.pallas import tpu as pltpu
from jax.experimental.pallas import tpu_sc as plsc

_N = 10000
_E = 320000
_H = 128
_L = 4
_NCENT = 32
_CUT = 6.0
_GAMMA = 10.0 / (_CUT - 0.0 + 1e-06) ** 2

_NW = 32
_CHUNK = 128
_CPW = 79
_EPW = _CHUNK * _CPW
_EP = _NW * _EPW
_NPAD = 10240

_EBLK = 2048
_NBLK = 2000


def _edge_mlp_body(d_ref, w1_ref, b1_ref, w2_ref, b2_ref, out_ref):
    centers = lax.broadcasted_iota(jnp.int32, (1, _NCENT), 1).astype(jnp.float32) * (
        _CUT / (_NCENT - 1))
    diff = d_ref[...] - centers
    rbf = jnp.exp((-_GAMMA) * diff * diff)
    h = jnp.dot(rbf, w1_ref[...], preferred_element_type=jnp.float32)
    h = h + b1_ref[...]
    h = h * jax.nn.sigmoid(h)
    e = jnp.dot(h, w2_ref[...], preferred_element_type=jnp.float32)
    out_ref[...] = e + b2_ref[...]


def _edge_mlp(d, w1, b1, w2, b2):
    grid = _EP // _EBLK
    return pl.pallas_call(
        _edge_mlp_body,
        grid=(grid,),
        in_specs=[
            pl.BlockSpec((_EBLK, 1), lambda i: (i, 0)),
            pl.BlockSpec((_NCENT, _H), lambda i: (0, 0)),
            pl.BlockSpec((1, _H), lambda i: (0, 0)),
            pl.BlockSpec((_H, _H), lambda i: (0, 0)),
            pl.BlockSpec((1, _H), lambda i: (0, 0)),
        ],
        out_specs=pl.BlockSpec((_EBLK, _H), lambda i: (i, 0)),
        out_shape=jax.ShapeDtypeStruct((_EP, _H), jnp.float32),
    )(d, w1, b1, w2, b2)


def _embed_body(z_ref, emb_ref, out_ref):
    ids = lax.broadcasted_iota(jnp.int32, (_NBLK, _H), 1)
    oh = (z_ref[...] == ids).astype(jnp.float32)
    out_ref[...] = jnp.dot(oh, emb_ref[...], preferred_element_type=jnp.float32)


def _embed(z2d, emb_pad):
    return pl.pallas_call(
        _embed_body,
        grid=(_N // _NBLK,),
        in_specs=[
            pl.BlockSpec((_NBLK, 1), lambda i: (i, 0)),
            pl.BlockSpec((_H, _H), lambda i: (0, 0)),
        ],
        out_specs=pl.BlockSpec((_NBLK, _H), lambda i: (i, 0)),
        out_shape=jax.ShapeDtypeStruct((_N, _H), jnp.float32),
    )(z2d, emb_pad)


def _node_mlp_body(x_ref, a0_ref, a1_ref, w1_ref, b1_ref, w2_ref, b2_ref, out_ref):
    h = x_ref[...] + a0_ref[...] + a1_ref[...]
    t = jnp.dot(h, w1_ref[...], preferred_element_type=jnp.float32) + b1_ref[...]
    t = t * jax.nn.sigmoid(t)
    o = jnp.dot(t, w2_ref[...], preferred_element_type=jnp.float32) + b2_ref[...]
    out_ref[...] = o * jax.nn.sigmoid(o)


def _node_mlp(x, a0, a1, w1, b1, w2, b2):
    return pl.pallas_call(
        _node_mlp_body,
        grid=(_N // _NBLK,),
        in_specs=[
            pl.BlockSpec((_NBLK, _H), lambda i: (i, 0)),
            pl.BlockSpec((_NBLK, _H), lambda i: (i, 0)),
            pl.BlockSpec((_NBLK, _H), lambda i: (i, 0)),
            pl.BlockSpec((_H, _H), lambda i: (0, 0)),
            pl.BlockSpec((1, _H), lambda i: (0, 0)),
            pl.BlockSpec((_H, _H), lambda i: (0, 0)),
            pl.BlockSpec((1, _H), lambda i: (0, 0)),
        ],
        out_specs=pl.BlockSpec((_NBLK, _H), lambda i: (i, 0)),
        out_shape=jax.ShapeDtypeStruct((_N, _H), jnp.float32),
    )(x, a0, a1, w1, b1, w2, b2)


def _head_body(x_ref, wp1_ref, bp1_ref, wp2_ref, bp2_ref, out_ref):
    g = jnp.mean(x_ref[...], axis=0, keepdims=True)
    t = jnp.dot(g, wp1_ref[...], preferred_element_type=jnp.float32) + bp1_ref[...]
    t = t * jax.nn.sigmoid(t)
    zz = jnp.dot(t, wp2_ref[...], preferred_element_type=jnp.float32) + bp2_ref[...]
    nrm = jnp.sqrt(jnp.sum(zz * zz, axis=-1, keepdims=True))
    out_ref[...] = zz / jnp.maximum(nrm, 1e-12)


def _head(x, wp1, bp1, wp2, bp2):
    return pl.pallas_call(
        _head_body,
        out_shape=jax.ShapeDtypeStruct((1, _H), jnp.float32),
    )(x, wp1, bp1, wp2, bp2)


_SC_MESH = plsc.VectorSubcoreMesh(core_axis_name="c", subcore_axis_name="s")


@functools.partial(
    pl.kernel,
    out_type=jax.ShapeDtypeStruct((2, _NPAD, _H), jnp.float32),
    mesh=_SC_MESH,
    scratch_types=[
        pltpu.VMEM((_CHUNK,), jnp.int32),
        pltpu.VMEM((_CHUNK,), jnp.int32),
        pltpu.VMEM((_CHUNK, _H), jnp.float32),
        pltpu.VMEM((_CHUNK, _H), jnp.float32),
        pltpu.VMEM_SHARED((_NPAD, _H), jnp.float32),
        pltpu.SemaphoreType.DMA,
    ],
)
def _sc_layer(src_hbm, dst_hbm, e_hbm, x_hbm, out_hbm,
              src_v, dst_v, e_v, x_v, acc_sh, sem):
    cid = lax.axis_index("c")
    sid = lax.axis_index("s")
    wid = sid * 2 + cid

    z16 = jnp.zeros((16,), jnp.float32)

    @pl.loop(0, _CHUNK)
    def _zrow(r):
        for j in range(8):
            e_v[r, pl.ds(j * 16, 16)] = z16

    for k in range(5):
        pltpu.sync_copy(e_v, acc_sh.at[pl.ds(sid * 640 + k * _CHUNK, _CHUNK), :])
    plsc.subcore_barrier()

    base0 = wid * _EPW

    @pl.loop(0, _CPW)
    def _chunk(c):
        base = base0 + c * _CHUNK
        pltpu.sync_copy(src_hbm.at[pl.ds(base, _CHUNK)], src_v)
        pltpu.sync_copy(dst_hbm.at[pl.ds(base, _CHUNK)], dst_v)
        pltpu.sync_copy(e_hbm.at[pl.ds(base, _CHUNK), :], e_v)
        pltpu.async_copy(x_hbm.at[src_v], x_v, sem).wait()

        @pl.loop(0, _CHUNK)
        def _crow(r):
            for j in range(8):
                s = pl.ds(j * 16, 16)
                e_v[r, s] = jnp.maximum(e_v[r, s] + x_v[r, s], 0.0)

        pltpu.sync_copy(e_v, acc_sh.at[dst_v], add=True)

    plsc.subcore_barrier()

    for k in range(5):
        r0 = sid * 640 + k * _CHUNK
        pltpu.sync_copy(acc_sh.at[pl.ds(r0, _CHUNK), :], e_v)
        pltpu.sync_copy(e_v, out_hbm.at[cid, pl.ds(r0, _CHUNK), :])


def kernel(z, edge_index, edge_attr, emb, We1, be1, We2, be2,
           Wm1, bm1, Wm2, bm2, Wp1, bp1, Wp2, bp2):
    src = jnp.pad(edge_index[0].astype(jnp.int32), (0, _EP - _E))
    dst = jnp.pad(edge_index[1].astype(jnp.int32), (0, _EP - _E),
                  constant_values=_N)
    d2 = jnp.pad(edge_attr.astype(jnp.float32), (0, _EP - _E)).reshape(_EP, 1)

    e = _edge_mlp(d2, We1, be1.reshape(1, _H), We2, be2.reshape(1, _H))

    emb_pad = jnp.pad(emb, ((0, _H - emb.shape[0]), (0, 0)))
    x = _embed(z.reshape(_N, 1).astype(jnp.int32), emb_pad)

    for i in range(_L):
        agg = _sc_layer(src, dst, e, x)
        x = _node_mlp(x, agg[0], agg[1], Wm1[i], bm1[i].reshape(1, _H),
                      Wm2[i], bm2[i].reshape(1, _H))

    return _head(x, Wp1, bp1.reshape(1, _H), Wp2, bp2.reshape(1, _H))
